# Initial kernel scaffold; baseline (speedup 1.0000x reference)
import jax
import jax.numpy as jnp
from jax.experimental import pallas as pl


def kernel(X, token_table, pos_embed):
    raise NotImplementedError("write your pallas kernel here")


# speedup vs baseline: 1.0654x; 1.0654x over previous
import functools

import jax
import jax.numpy as jnp
from jax import lax
from jax.experimental import pallas as pl
from jax.experimental.pallas import tpu as pltpu
from jax.experimental.pallas import tpu_sc as plsc

_info = plsc.get_sparse_core_info()
_NC, _NS, _L = _info.num_cores, _info.num_subcores, _info.num_lanes
_NW = _NC * _NS

_D = 128
_CHUNK = 128
_CPW = 2


def _build(n_rows, vocab, d):
    assert n_rows == _NW * _CPW * _CHUNK and d == _D
    mesh = plsc.VectorSubcoreMesh(core_axis_name="c", subcore_axis_name="s")

    @functools.partial(
        pl.kernel,
        mesh=mesh,
        out_type=jax.ShapeDtypeStruct((n_rows // _CHUNK, _CHUNK, d), jnp.float32),
        scratch_types=[
            pltpu.VMEM((_CPW, _CHUNK), jnp.int32),
            pltpu.VMEM((_CPW, _CHUNK, d), jnp.float32),
            pltpu.VMEM((_CPW, _CHUNK, d), jnp.float32),
            pltpu.SemaphoreType.DMA,
        ],
    )
    def k(x_hbm, table_hbm, pos_hbm, out_hbm, idx_v, rows_v, pos_v, sem):
        n_pos_chunks = pos_hbm.shape[0]
        wid = lax.axis_index("s") * _NC + lax.axis_index("c")
        pltpu.sync_copy(x_hbm.at[pl.ds(wid * _CPW, _CPW)], idx_v)
        cps = [
            pltpu.async_copy(table_hbm.at[idx_v.at[j]], rows_v.at[j], sem)
            for j in range(_CPW)
        ]
        pos_chunk = (wid * _CPW) % n_pos_chunks
        pltpu.sync_copy(pos_hbm.at[pl.ds(pos_chunk, _CPW)], pos_v)
        for cp in cps:
            cp.wait()

        def body(r, carry):
            for j in range(_CPW):
                for c in range(d // _L):
                    sl = pl.ds(c * _L, _L)
                    rows_v[j, r, sl] = rows_v[j, r, sl] + pos_v[j, r, sl]
            return carry

        lax.fori_loop(0, _CHUNK, body, 0, unroll=2)
        pltpu.sync_copy(rows_v, out_hbm.at[pl.ds(wid * _CPW, _CPW)])

    return k


def kernel(X, token_table, pos_embed):
    b, s = X.shape
    vocab, d = token_table.shape
    n = b * s
    xf = X.reshape(n // _CHUNK, _CHUNK).astype(jnp.int32)
    pos3 = pos_embed.reshape(s // _CHUNK, _CHUNK, d).astype(jnp.float32)
    out = _build(n, vocab, d)(xf, token_table, pos3)
    return out.reshape(b, s, d)


# speedup vs baseline: 1.1294x; 1.0600x over previous
import functools

import jax
import jax.numpy as jnp
from jax import lax
from jax.experimental import pallas as pl
from jax.experimental.pallas import tpu as pltpu
from jax.experimental.pallas import tpu_sc as plsc

_info = plsc.get_sparse_core_info()
_NC, _NS, _L = _info.num_cores, _info.num_subcores, _info.num_lanes
_NW = _NC * _NS

_D = 128
_CHUNK = 128
_CPW = 2


def _build(n_rows, vocab, d):
    assert n_rows == _NW * _CPW * _CHUNK and d == _D
    mesh = plsc.VectorSubcoreMesh(core_axis_name="c", subcore_axis_name="s")

    @functools.partial(
        pl.kernel,
        mesh=mesh,
        out_type=jax.ShapeDtypeStruct((n_rows // _CHUNK, _CHUNK, d), jnp.float32),
        scratch_types=[
            pltpu.VMEM((_CPW, _CHUNK), jnp.int32),
            pltpu.VMEM((_CPW, _CHUNK, d), jnp.float32),
            pltpu.SemaphoreType.DMA,
        ],
    )
    def k(x_hbm, table_hbm, pos_hbm, out_hbm, idx_v, rows_v, sem):
        n_pos_chunks = pos_hbm.shape[0]
        wid = lax.axis_index("s") * _NC + lax.axis_index("c")
        pltpu.sync_copy(x_hbm.at[pl.ds(wid * _CPW, _CPW)], idx_v)
        pos_chunk = (wid * _CPW) % n_pos_chunks
        pltpu.sync_copy(pos_hbm.at[pl.ds(pos_chunk, _CPW)], rows_v)
        cps = [
            pltpu.async_copy(
                table_hbm.at[idx_v.at[j]], rows_v.at[j], sem, add=True
            )
            for j in range(_CPW)
        ]
        for cp in cps:
            cp.wait()
        pltpu.sync_copy(rows_v, out_hbm.at[pl.ds(wid * _CPW, _CPW)])

    return k


def kernel(X, token_table, pos_embed):
    b, s = X.shape
    vocab, d = token_table.shape
    n = b * s
    xf = X.reshape(n // _CHUNK, _CHUNK).astype(jnp.int32)
    pos3 = pos_embed.reshape(s // _CHUNK, _CHUNK, d).astype(jnp.float32)
    out = _build(n, vocab, d)(xf, token_table, pos3)
    return out.reshape(b, s, d)


# speedup vs baseline: 1.1559x; 1.0235x over previous
import functools

import jax
import jax.numpy as jnp
from jax import lax
from jax.experimental import pallas as pl
from jax.experimental.pallas import tpu as pltpu
from jax.experimental.pallas import tpu_sc as plsc

_info = plsc.get_sparse_core_info()
_NC, _NS, _L = _info.num_cores, _info.num_subcores, _info.num_lanes
_NW = _NC * _NS

_D = 128
_CHUNK = 64
_CPW = 4


def _build(n_rows, d):
    assert n_rows == _NW * _CPW * _CHUNK and d == _D
    mesh = plsc.VectorSubcoreMesh(core_axis_name="c", subcore_axis_name="s")

    @functools.partial(
        pl.kernel,
        mesh=mesh,
        out_type=jax.ShapeDtypeStruct((n_rows // _CHUNK, _CHUNK, d), jnp.float32),
        scratch_types=[
            pltpu.VMEM((_CPW, _CHUNK), jnp.int32),
            pltpu.VMEM((_CPW, _CHUNK, d), jnp.float32),
            pltpu.SemaphoreType.DMA,
            pltpu.SemaphoreType.DMA,
            pltpu.SemaphoreType.DMA,
        ],
    )
    def k(x_hbm, table_hbm, pos_hbm, out_hbm, idx_v, rows_v, sem_p, sem_g, sem_w):
        n_pos_chunks = pos_hbm.shape[0]
        wid = lax.axis_index("s") * _NC + lax.axis_index("c")
        base = wid * _CPW
        pltpu.sync_copy(x_hbm.at[pl.ds(base, _CPW)], idx_v)
        pos_cps = [
            pltpu.async_copy(
                pos_hbm.at[(base + j) % n_pos_chunks], rows_v.at[j], sem_p
            )
            for j in range(_CPW)
        ]
        g_cps = []
        for j in range(_CPW):
            pos_cps[j].wait()
            g_cps.append(
                pltpu.async_copy(
                    table_hbm.at[idx_v.at[j]], rows_v.at[j], sem_g, add=True
                )
            )
        w_cps = []
        for j in range(_CPW):
            g_cps[j].wait()
            w_cps.append(
                pltpu.async_copy(rows_v.at[j], out_hbm.at[base + j], sem_w)
            )
        for cp in w_cps:
            cp.wait()

    return k


def kernel(X, token_table, pos_embed):
    b, s = X.shape
    vocab, d = token_table.shape
    n = b * s
    xf = X.reshape(n // _CHUNK, _CHUNK).astype(jnp.int32)
    pos3 = pos_embed.reshape(s // _CHUNK, _CHUNK, d).astype(jnp.float32)
    out = _build(n, d)(xf, token_table, pos3)
    return out.reshape(b, s, d)
